# baseline (speedup 1.0000x reference)
import functools

import numpy as np
import jax
import jax.numpy as jnp
from jax import lax
from jax.experimental import pallas as pl
from jax.experimental.pallas import tpu as pltpu
from jax.experimental.pallas import tpu_sc as plsc

_CATEGORICAL_P = 0.1
_ORDERED_P = 0.7
_IDX_CHUNK = 128


def _draw_num_classes() -> int:
    xs = np.arange(1, 10)
    w = xs ** (-0.8)
    w = w / w.sum()
    return int(np.random.default_rng(0).choice(xs, p=w))


@functools.lru_cache(maxsize=None)
def _constants(S, B, H):
    nc = _draw_num_classes()
    kk = max(nc - 1, 0)
    with jax.ensure_compile_time_eval(), \
            jax.default_device(jax.local_devices(backend="cpu")[0]):
        key = jax.random.key(42)
        k1, k2, k3, k4 = jax.random.split(key, 4)
        cat = np.asarray(jax.random.uniform(k1, (B, H)) < _CATEGORICAL_P)
        if kk:
            ind = np.asarray(jax.random.randint(k2, (kk, B, H), 0, S))
        else:
            ind = np.zeros((0, B, H), np.int32)
        orda = np.asarray(jax.random.uniform(k3, (B, H)) < _ORDERED_P) & cat
        perm = np.asarray(jax.random.permutation(k4, nc)).astype(np.float64)
    cat_only = cat & ~orda

    def f_ord(cnt):
        code = cnt - nc / 2.0
        if nc % 2 == 0 and code == int(code) and 0 <= code < nc:
            return float(perm[int(code)])
        return 0.0

    coefs = np.zeros((kk + 2, B, H), np.float32)
    coefs[0] = (~cat).astype(np.float32)
    coefs[1] = np.where(cat_only, -nc / 2.0, np.where(orda, f_ord(0), 0.0))
    for j in range(1, kk + 1):
        dj = f_ord(j) - f_ord(j - 1)
        coefs[j + 1] = np.where(cat_only, 1.0, np.where(orda, dj, 0.0))

    C = B * H
    cs = np.arange(C, dtype=np.int32)
    rows = (ind.reshape(kk, C).astype(np.int32) * B + (cs // H)[None, :]).reshape(-1)
    cols = np.broadcast_to((cs % H)[None, :], (kk, C)).reshape(-1).astype(np.int32)
    return nc, kk, coefs, rows, cols


def _sc_gather(x2, rows3d, cols3d):
    NW, J, L = rows3d.shape
    H = x2.shape[1]
    info = plsc.get_sparse_core_info()
    ncores = info.num_cores
    mesh = plsc.VectorSubcoreMesh(core_axis_name="c", subcore_axis_name="s")

    @functools.partial(
        pl.kernel,
        mesh=mesh,
        out_type=jax.ShapeDtypeStruct((NW, J, L), jnp.float32),
        compiler_params=pltpu.CompilerParams(needs_layout_passes=False),
        scratch_types=[
            pltpu.VMEM((J, L), jnp.int32),
            pltpu.VMEM((J, L), jnp.int32),
            pltpu.VMEM((J, L), jnp.float32),
            pltpu.VMEM((2, L, H), jnp.float32),
            pltpu.SemaphoreType.DMA,
            pltpu.SemaphoreType.DMA,
        ],
    )
    def gather_kernel(x_hbm, rows_hbm, cols_hbm, out_hbm,
                      rid_v, cid_v, vals_v, rbuf, sem0, sem1):
        wid = lax.axis_index("s") * ncores + lax.axis_index("c")
        pltpu.sync_copy(rows_hbm.at[wid], rid_v)
        pltpu.sync_copy(cols_hbm.at[wid], cid_v)
        sems = (sem0, sem1)
        handles = [None, None]
        handles[0] = pltpu.async_copy(
            x_hbm.at[rid_v.at[0]], rbuf.at[0], sems[0])
        for j in range(J):
            cur = j % 2
            if j + 1 < J:
                handles[1 - cur] = pltpu.async_copy(
                    x_hbm.at[rid_v.at[j + 1]], rbuf.at[1 - cur], sems[1 - cur])
            handles[cur].wait()
            for g in range(L // 16):
                rows16 = lax.iota(jnp.int32, 16) + (g * 16)
                cols16 = cid_v[j, pl.ds(g * 16, 16)]
                vals_v[j, pl.ds(g * 16, 16)] = plsc.load_gather(
                    rbuf.at[cur], [rows16, cols16])
        pltpu.sync_copy(vals_v, out_hbm.at[wid])

    return gather_kernel(x2, jnp.asarray(rows3d), jnp.asarray(cols3d))


def _dense_pass(x, bvals, coefs, nc, kk, block_s=128):
    S, B, H = x.shape

    def body(x_ref, b_ref, c_ref, o_ref):
        xb = x_ref[...]
        y = xb / (1.0 + jnp.abs(xb))
        cf = c_ref[...]
        out = cf[0:1] * y + cf[1:2]
        if kk:
            b = b_ref[...]
            rows = [b[k:k + 1] for k in range(kk)]
            for i in range(kk):
                for j2 in range(kk - 1 - i):
                    a_, c_ = rows[j2], rows[j2 + 1]
                    rows[j2] = jnp.minimum(a_, c_)
                    rows[j2 + 1] = jnp.maximum(a_, c_)
            ss = [r / (1.0 + jnp.abs(r)) for r in rows]
            for j in range(1, kk + 1):
                out = out + jnp.where(y > ss[j - 1], cf[j + 1:j + 2], 0.0)
        o_ref[...] = out

    bk = max(kk, 1)
    bh = H // 2
    return pl.pallas_call(
        body,
        grid=(S // block_s, 2),
        in_specs=[
            pl.BlockSpec((block_s, B, bh), lambda i, j: (i, 0, j)),
            pl.BlockSpec((bk, B, bh), lambda i, j: (0, 0, j)),
            pl.BlockSpec((kk + 2, B, bh), lambda i, j: (0, 0, j)),
        ],
        out_specs=pl.BlockSpec((block_s, B, bh), lambda i, j: (i, 0, j)),
        out_shape=jax.ShapeDtypeStruct((S, B, H), jnp.float32),
    )(x, bvals, coefs)


def kernel(x):
    S, B, H = x.shape
    nc, kk, coefs_np, rows_np, cols_np = _constants(S, B, H)
    C = B * H
    coefs = jnp.asarray(coefs_np)

    if kk:
        info = plsc.get_sparse_core_info()
        nw = info.num_cores * info.num_subcores
        total = kk * C
        chunk = nw * _IDX_CHUNK
        padded = ((total + chunk - 1) // chunk) * chunk
        rows = np.arange(padded, dtype=np.int32) % (S * B)
        cols = np.zeros(padded, np.int32)
        rows[:total] = rows_np
        cols[:total] = cols_np
        rows3d = rows.reshape(nw, padded // chunk, _IDX_CHUNK)
        cols3d = cols.reshape(nw, padded // chunk, _IDX_CHUNK)
        gathered = _sc_gather(x.reshape(S * B, H), rows3d, cols3d)
        bvals = gathered.reshape(-1)[:total].reshape(kk, B, H)
    else:
        bvals = jnp.zeros((1, B, H), jnp.float32)

    return _dense_pass(x, bvals, coefs, nc, kk)


# speedup vs baseline: 1.0698x; 1.0698x over previous
import functools

import numpy as np
import jax
import jax.numpy as jnp
from jax import lax
from jax.experimental import pallas as pl
from jax.experimental.pallas import tpu as pltpu
from jax.experimental.pallas import tpu_sc as plsc

_CATEGORICAL_P = 0.1
_ORDERED_P = 0.7
_IDX_CHUNK = 128


def _draw_num_classes() -> int:
    xs = np.arange(1, 10)
    w = xs ** (-0.8)
    w = w / w.sum()
    return int(np.random.default_rng(0).choice(xs, p=w))


@functools.lru_cache(maxsize=None)
def _constants(S, B, H):
    nc = _draw_num_classes()
    kk = max(nc - 1, 0)
    with jax.ensure_compile_time_eval(), \
            jax.default_device(jax.local_devices(backend="cpu")[0]):
        key = jax.random.key(42)
        k1, k2, k3, k4 = jax.random.split(key, 4)
        cat = np.asarray(jax.random.uniform(k1, (B, H)) < _CATEGORICAL_P)
        if kk:
            ind = np.asarray(jax.random.randint(k2, (kk, B, H), 0, S))
        else:
            ind = np.zeros((0, B, H), np.int32)
        orda = np.asarray(jax.random.uniform(k3, (B, H)) < _ORDERED_P) & cat
        perm = np.asarray(jax.random.permutation(k4, nc)).astype(np.float64)
    cat_only = cat & ~orda

    def f_ord(cnt):
        code = cnt - nc / 2.0
        if nc % 2 == 0 and code == int(code) and 0 <= code < nc:
            return float(perm[int(code)])
        return 0.0

    coefs = np.zeros((kk + 2, B, H), np.float32)
    coefs[0] = (~cat).astype(np.float32)
    coefs[1] = np.where(cat_only, -nc / 2.0, np.where(orda, f_ord(0), 0.0))
    for j in range(1, kk + 1):
        dj = f_ord(j) - f_ord(j - 1)
        coefs[j + 1] = np.where(cat_only, 1.0, np.where(orda, dj, 0.0))

    C = B * H
    cs = np.arange(C, dtype=np.int32)
    rows = (ind.reshape(kk, C).astype(np.int32) * B + (cs // H)[None, :]).reshape(-1)
    cols = np.broadcast_to((cs % H)[None, :], (kk, C)).reshape(-1).astype(np.int32)
    return nc, kk, coefs, rows, cols


def _sc_gather(x2, rows3d, cols3d):
    NW, J, L = rows3d.shape
    H = x2.shape[1]
    info = plsc.get_sparse_core_info()
    ncores = info.num_cores
    mesh = plsc.VectorSubcoreMesh(core_axis_name="c", subcore_axis_name="s")

    @functools.partial(
        pl.kernel,
        mesh=mesh,
        out_type=jax.ShapeDtypeStruct((NW, J, L), jnp.float32),
        compiler_params=pltpu.CompilerParams(needs_layout_passes=False),
        scratch_types=[
            pltpu.VMEM((J, L), jnp.int32),
            pltpu.VMEM((J, L), jnp.int32),
            pltpu.VMEM((J, L), jnp.float32),
            pltpu.VMEM((2, L, H), jnp.float32),
            pltpu.SemaphoreType.DMA,
            pltpu.SemaphoreType.DMA,
        ],
    )
    def gather_kernel(x_hbm, rows_hbm, cols_hbm, out_hbm,
                      rid_v, cid_v, vals_v, rbuf, sem0, sem1):
        wid = lax.axis_index("s") * ncores + lax.axis_index("c")
        pltpu.sync_copy(rows_hbm.at[wid], rid_v)
        pltpu.sync_copy(cols_hbm.at[wid], cid_v)
        sems = (sem0, sem1)
        handles = [None, None]
        handles[0] = pltpu.async_copy(
            x_hbm.at[rid_v.at[0]], rbuf.at[0], sems[0])
        for j in range(J):
            cur = j % 2
            if j + 1 < J:
                handles[1 - cur] = pltpu.async_copy(
                    x_hbm.at[rid_v.at[j + 1]], rbuf.at[1 - cur], sems[1 - cur])
            handles[cur].wait()
            for g in range(L // 16):
                rows16 = lax.iota(jnp.int32, 16) + (g * 16)
                cols16 = cid_v[j, pl.ds(g * 16, 16)]
                vals_v[j, pl.ds(g * 16, 16)] = plsc.load_gather(
                    rbuf.at[cur], [rows16, cols16])
        pltpu.sync_copy(vals_v, out_hbm.at[wid])

    return gather_kernel(x2, jnp.asarray(rows3d), jnp.asarray(cols3d))


def _dense_pass(x, bvals, coefs, nc, kk, block_s=128):
    S, B, H = x.shape

    def body(x_ref, b_ref, c_ref, o_ref):
        xb = x_ref[...]
        y = xb / (1.0 + jnp.abs(xb))
        cf = c_ref[...]
        out = cf[0:1] * y + cf[1:2]
        if kk:
            b = b_ref[...]
            rows = [b[k:k + 1] for k in range(kk)]
            for i in range(kk):
                for j2 in range(kk - 1 - i):
                    a_, c_ = rows[j2], rows[j2 + 1]
                    rows[j2] = jnp.minimum(a_, c_)
                    rows[j2 + 1] = jnp.maximum(a_, c_)
            ss = [r / (1.0 + jnp.abs(r)) for r in rows]
            for j in range(1, kk + 1):
                out = out + jnp.where(y > ss[j - 1], cf[j + 1:j + 2], 0.0)
        o_ref[...] = out

    bk = max(kk, 1)
    return pl.pallas_call(
        body,
        grid=(S // block_s,),
        in_specs=[
            pl.BlockSpec((block_s, B, H), lambda i: (i, 0, 0)),
            pl.BlockSpec((bk, B, H), lambda i: (0, 0, 0)),
            pl.BlockSpec((kk + 2, B, H), lambda i: (0, 0, 0)),
        ],
        out_specs=pl.BlockSpec((block_s, B, H), lambda i: (i, 0, 0)),
        out_shape=jax.ShapeDtypeStruct((S, B, H), jnp.float32),
    )(x, bvals, coefs)


def kernel(x):
    S, B, H = x.shape
    nc, kk, coefs_np, rows_np, cols_np = _constants(S, B, H)
    C = B * H
    coefs = jnp.asarray(coefs_np)

    if kk:
        info = plsc.get_sparse_core_info()
        nw = info.num_cores * info.num_subcores
        total = kk * C
        chunk = nw * _IDX_CHUNK
        padded = ((total + chunk - 1) // chunk) * chunk
        rows = np.arange(padded, dtype=np.int32) % (S * B)
        cols = np.zeros(padded, np.int32)
        rows[:total] = rows_np
        cols[:total] = cols_np
        rows3d = rows.reshape(nw, padded // chunk, _IDX_CHUNK)
        cols3d = cols.reshape(nw, padded // chunk, _IDX_CHUNK)
        gathered = _sc_gather(x.reshape(S * B, H), rows3d, cols3d)
        bvals = gathered.reshape(-1)[:total].reshape(kk, B, H)
    else:
        bvals = jnp.zeros((1, B, H), jnp.float32)

    return _dense_pass(x, bvals, coefs, nc, kk)


# speedup vs baseline: 1.0846x; 1.0138x over previous
import functools

import numpy as np
import jax
import jax.numpy as jnp
from jax import lax
from jax.experimental import pallas as pl
from jax.experimental.pallas import tpu as pltpu
from jax.experimental.pallas import tpu_sc as plsc

_CATEGORICAL_P = 0.1
_ORDERED_P = 0.7
_IDX_CHUNK = 128


def _draw_num_classes() -> int:
    xs = np.arange(1, 10)
    w = xs ** (-0.8)
    w = w / w.sum()
    return int(np.random.default_rng(0).choice(xs, p=w))


@functools.lru_cache(maxsize=None)
def _constants(S, B, H):
    nc = _draw_num_classes()
    kk = max(nc - 1, 0)
    with jax.ensure_compile_time_eval(), \
            jax.default_device(jax.local_devices(backend="cpu")[0]):
        key = jax.random.key(42)
        k1, k2, k3, k4 = jax.random.split(key, 4)
        cat = np.asarray(jax.random.uniform(k1, (B, H)) < _CATEGORICAL_P)
        if kk:
            ind = np.asarray(jax.random.randint(k2, (kk, B, H), 0, S))
        else:
            ind = np.zeros((0, B, H), np.int32)
        orda = np.asarray(jax.random.uniform(k3, (B, H)) < _ORDERED_P) & cat
        perm = np.asarray(jax.random.permutation(k4, nc)).astype(np.float64)
    cat_only = cat & ~orda

    def f_ord(cnt):
        code = cnt - nc / 2.0
        if nc % 2 == 0 and code == int(code) and 0 <= code < nc:
            return float(perm[int(code)])
        return 0.0

    coefs = np.zeros((kk + 2, B, H), np.float32)
    coefs[0] = (~cat).astype(np.float32)
    coefs[1] = np.where(cat_only, -nc / 2.0, np.where(orda, f_ord(0), 0.0))
    for j in range(1, kk + 1):
        dj = f_ord(j) - f_ord(j - 1)
        coefs[j + 1] = np.where(cat_only, 1.0, np.where(orda, dj, 0.0))

    C = B * H
    cs = np.arange(C, dtype=np.int32)
    rows = (ind.reshape(kk, C).astype(np.int32) * B + (cs // H)[None, :]).reshape(-1)
    cols = np.broadcast_to((cs % H)[None, :], (kk, C)).reshape(-1).astype(np.int32)
    return nc, kk, coefs, rows, cols


def _sc_gather(x2, rows3d, cols3d):
    NW, J, L = rows3d.shape
    H = x2.shape[1]
    info = plsc.get_sparse_core_info()
    ncores = info.num_cores
    mesh = plsc.VectorSubcoreMesh(core_axis_name="c", subcore_axis_name="s")

    @functools.partial(
        pl.kernel,
        mesh=mesh,
        out_type=jax.ShapeDtypeStruct((NW, J, L), jnp.float32),
        compiler_params=pltpu.CompilerParams(needs_layout_passes=False),
        scratch_types=[
            pltpu.VMEM((J, L), jnp.int32),
            pltpu.VMEM((J, L), jnp.int32),
            pltpu.VMEM((J, L), jnp.float32),
            pltpu.VMEM((3, L, H), jnp.float32),
            pltpu.SemaphoreType.DMA,
            pltpu.SemaphoreType.DMA,
            pltpu.SemaphoreType.DMA,
        ],
    )
    def gather_kernel(x_hbm, rows_hbm, cols_hbm, out_hbm,
                      rid_v, cid_v, vals_v, rbuf, sem0, sem1, sem2):
        wid = lax.axis_index("s") * ncores + lax.axis_index("c")
        pltpu.sync_copy(rows_hbm.at[wid], rid_v)
        pltpu.sync_copy(cols_hbm.at[wid], cid_v)
        nbuf = 3
        sems = (sem0, sem1, sem2)
        handles = [None] * nbuf
        for j0 in range(min(nbuf - 1, J)):
            handles[j0] = pltpu.async_copy(
                x_hbm.at[rid_v.at[j0]], rbuf.at[j0], sems[j0])
        for j in range(J):
            cur = j % nbuf
            nxt = j + nbuf - 1
            if nxt < J:
                handles[nxt % nbuf] = pltpu.async_copy(
                    x_hbm.at[rid_v.at[nxt]], rbuf.at[nxt % nbuf],
                    sems[nxt % nbuf])
            handles[cur].wait()
            for g in range(L // 16):
                rows16 = lax.iota(jnp.int32, 16) + (g * 16)
                cols16 = cid_v[j, pl.ds(g * 16, 16)]
                vals_v[j, pl.ds(g * 16, 16)] = plsc.load_gather(
                    rbuf.at[cur], [rows16, cols16])
        pltpu.sync_copy(vals_v, out_hbm.at[wid])

    return gather_kernel(x2, jnp.asarray(rows3d), jnp.asarray(cols3d))


def _dense_pass(x, bvals, coefs, nc, kk, block_s=128):
    S, B, H = x.shape

    def body(x_ref, b_ref, c_ref, o_ref):
        xb = x_ref[...]
        y = xb / (1.0 + jnp.abs(xb))
        cf = c_ref[...]
        out = cf[0:1] * y + cf[1:2]
        if kk:
            b = b_ref[...]
            rows = [b[k:k + 1] for k in range(kk)]
            for i in range(kk):
                for j2 in range(kk - 1 - i):
                    a_, c_ = rows[j2], rows[j2 + 1]
                    rows[j2] = jnp.minimum(a_, c_)
                    rows[j2 + 1] = jnp.maximum(a_, c_)
            ss = [r / (1.0 + jnp.abs(r)) for r in rows]
            for j in range(1, kk + 1):
                out = out + jnp.where(y > ss[j - 1], cf[j + 1:j + 2], 0.0)
        o_ref[...] = out

    bk = max(kk, 1)
    return pl.pallas_call(
        body,
        grid=(S // block_s,),
        in_specs=[
            pl.BlockSpec((block_s, B, H), lambda i: (i, 0, 0)),
            pl.BlockSpec((bk, B, H), lambda i: (0, 0, 0)),
            pl.BlockSpec((kk + 2, B, H), lambda i: (0, 0, 0)),
        ],
        out_specs=pl.BlockSpec((block_s, B, H), lambda i: (i, 0, 0)),
        out_shape=jax.ShapeDtypeStruct((S, B, H), jnp.float32),
    )(x, bvals, coefs)


def kernel(x):
    S, B, H = x.shape
    nc, kk, coefs_np, rows_np, cols_np = _constants(S, B, H)
    C = B * H
    coefs = jnp.asarray(coefs_np)

    if kk:
        info = plsc.get_sparse_core_info()
        nw = info.num_cores * info.num_subcores
        total = kk * C
        chunk = nw * _IDX_CHUNK
        padded = ((total + chunk - 1) // chunk) * chunk
        rows = np.arange(padded, dtype=np.int32) % (S * B)
        cols = np.zeros(padded, np.int32)
        rows[:total] = rows_np
        cols[:total] = cols_np
        rows3d = rows.reshape(nw, padded // chunk, _IDX_CHUNK)
        cols3d = cols.reshape(nw, padded // chunk, _IDX_CHUNK)
        gathered = _sc_gather(x.reshape(S * B, H), rows3d, cols3d)
        bvals = gathered.reshape(-1)[:total].reshape(kk, B, H)
    else:
        bvals = jnp.zeros((1, B, H), jnp.float32)

    return _dense_pass(x, bvals, coefs, nc, kk)
